# baseline (speedup 1.0000x reference)
import jax
import jax.numpy as jnp
from jax import lax
from jax.experimental import pallas as pl
from jax.experimental.pallas import tpu as pltpu
from jax.experimental.pallas import tpu_sc as plsc

NUM_TAGS = 64
BATCH_N = 1024
SEQ_N = 512
NUM_WORKERS = 32
B_PER_TILE = BATCH_N // NUM_WORKERS
LANES = 16
TVEC = SEQ_N // LANES

B_TC = 640
EM_BB = 64
EB_SC = (BATCH_N - B_TC) // NUM_WORKERS
ETAG_W = EB_SC + (-EB_SC) % 8
for _w in range(NUM_WORKERS):
    _e = B_TC + _w * EB_SC
    assert _e % 8 + EB_SC <= ETAG_W and _e - _e % 8 + ETAG_W <= BATCH_N


def _sc_body(tags_ref, em_ref, start_ref, end_ref, trans_ref, out_ref,
             tags_v, etags_v, slab0, slab1, slab2, start_v, end_v, trans_v,
             acc_v, sem0, sem1, sem2):
    cid = lax.axis_index("c")
    sid = lax.axis_index("s")
    wid = sid * 2 + cid
    iota = lax.iota(jnp.int32, LANES)
    zero16 = jnp.zeros((LANES,), jnp.int32)

    eb0 = B_TC + wid * EB_SC
    slabs = (slab0, slab1, slab2)
    sems = (sem0, sem1, sem2)
    NBUF = len(slabs)
    copies = []
    if EB_SC:
        copies = [pltpu.async_copy(em_ref.at[eb0 + i], slabs[i], sems[i])
                  for i in range(min(NBUF, EB_SC))]

    pltpu.sync_copy(tags_ref.at[pl.ds(wid * B_PER_TILE, B_PER_TILE)], tags_v)
    e_off = lax.rem(eb0, 8)
    if EB_SC:
        e_base = pl.multiple_of(eb0 - e_off, 8)
        pltpu.sync_copy(tags_ref.at[pl.ds(e_base, ETAG_W)], etags_v)
    pltpu.sync_copy(start_ref, start_v)
    pltpu.sync_copy(end_ref, end_v)
    pltpu.sync_copy(trans_ref, trans_v)

    acc = jnp.zeros((LANES,), jnp.float32)
    for h in range(B_PER_TILE // LANES):
        brow = h * LANES + iota
        first = plsc.load_gather(tags_v, [brow, zero16])
        last = plsc.load_gather(tags_v, [brow, zero16 + (SEQ_N - 1)])
        acc = acc + plsc.load_gather(start_v, [first])
        acc = acc + plsc.load_gather(end_v, [last])

    def row_body(b, acc):
        brow = jnp.broadcast_to(b, (LANES,))
        cur0 = tags_v[b, pl.ds(0, LANES)]
        prev0 = plsc.load_gather(tags_v, [brow, jnp.maximum(iota - 1, 0)])
        tv0 = plsc.load_gather(trans_v, [prev0 * NUM_TAGS + cur0])
        acc = acc + jnp.where(iota == 0, jnp.zeros_like(tv0), tv0)
        for j in range(1, TVEC):
            cur = tags_v[b, pl.ds(j * LANES, LANES)]
            prev = plsc.load_gather(tags_v, [brow, j * LANES - 1 + iota])
            acc = acc + plsc.load_gather(trans_v, [prev * NUM_TAGS + cur])
        return acc

    for k in range(max(EB_SC, 1)):
        r_lo = k * B_PER_TILE // max(EB_SC, 1)
        r_hi = (k + 1) * B_PER_TILE // max(EB_SC, 1)
        acc = lax.fori_loop(r_lo, r_hi, row_body, acc)
        if not EB_SC:
            continue
        buf = slabs[k % NBUF]
        copies[k].wait()
        for j in range(TVEC):
            cur = etags_v[e_off + k, pl.ds(j * LANES, LANES)]
            acc = acc + plsc.load_gather(buf, [cur, j * LANES + iota])
        if k + NBUF < EB_SC:
            copies.append(
                pltpu.async_copy(em_ref.at[eb0 + k + NBUF], buf,
                                 sems[k % NBUF]))
        else:
            copies.append(None)

    acc_v[...] = acc
    pltpu.sync_copy(acc_v, out_ref.at[wid])


def _em_sum_body(tags_ref, em_ref, out_ref):
    i = pl.program_id(0)
    t_blk = tags_ref[...]
    em_blk = em_ref[...]
    g = lax.broadcasted_iota(jnp.int32, em_blk.shape, 1)
    s = jnp.sum(jnp.where(g == t_blk[:, None, :], em_blk, 0.0))

    @pl.when(i == 0)
    def _init():
        out_ref[0, 0] = 0.0

    out_ref[0, 0] += s


@jax.jit
def _crf_score(em, tags_i32, start, end, trans_flat):
    em_t = jnp.transpose(em, (0, 2, 1))

    sc_part = pl.kernel(
        _sc_body,
        out_type=jax.ShapeDtypeStruct((NUM_WORKERS, LANES), jnp.float32),
        mesh=plsc.VectorSubcoreMesh(core_axis_name="c", subcore_axis_name="s"),
        compiler_params=pltpu.CompilerParams(needs_layout_passes=False),
        scratch_types=[
            pltpu.VMEM((B_PER_TILE, SEQ_N), jnp.int32),
            pltpu.VMEM((max(ETAG_W, 1), SEQ_N), jnp.int32),
            pltpu.VMEM((NUM_TAGS, SEQ_N), jnp.float32),
            pltpu.VMEM((NUM_TAGS, SEQ_N), jnp.float32),
            pltpu.VMEM((NUM_TAGS, SEQ_N), jnp.float32),
            pltpu.VMEM((NUM_TAGS,), jnp.float32),
            pltpu.VMEM((NUM_TAGS,), jnp.float32),
            pltpu.VMEM((NUM_TAGS * NUM_TAGS,), jnp.float32),
            pltpu.VMEM((LANES,), jnp.float32),
            pltpu.SemaphoreType.DMA,
            pltpu.SemaphoreType.DMA,
            pltpu.SemaphoreType.DMA,
        ],
    )
    partials = sc_part(tags_i32, em_t, start, end, trans_flat)

    em_sum = pl.pallas_call(
        _em_sum_body,
        grid=(B_TC // EM_BB,),
        in_specs=[
            pl.BlockSpec((EM_BB, SEQ_N), lambda i: (i, 0)),
            pl.BlockSpec((EM_BB, NUM_TAGS, SEQ_N), lambda i: (i, 0, 0)),
        ],
        out_specs=pl.BlockSpec((1, 1), lambda i: (0, 0),
                               memory_space=pltpu.SMEM),
        out_shape=jax.ShapeDtypeStruct((1, 1), jnp.float32),
        compiler_params=pltpu.CompilerParams(
            dimension_semantics=("arbitrary",)),
    )(tags_i32, em_t)

    return jnp.sum(partials) + em_sum[0, 0]


def kernel(emissions, tags, mask, start_transitions, end_transitions, transitions):
    del mask
    total = _crf_score(emissions, tags.astype(jnp.int32), start_transitions,
                       end_transitions, transitions.reshape(-1))
    return total / BATCH_N


# speedup vs baseline: 1.0094x; 1.0094x over previous
import jax
import jax.numpy as jnp
from jax import lax
from jax.experimental import pallas as pl
from jax.experimental.pallas import tpu as pltpu
from jax.experimental.pallas import tpu_sc as plsc

NUM_TAGS = 64
BATCH_N = 1024
SEQ_N = 512
NUM_WORKERS = 32
B_PER_TILE = BATCH_N // NUM_WORKERS
LANES = 16
TVEC = SEQ_N // LANES

B_TC = 768
EM_BB = 64
EB_SC = (BATCH_N - B_TC) // NUM_WORKERS
ETAG_W = EB_SC + (-EB_SC) % 8
for _w in range(NUM_WORKERS):
    _e = B_TC + _w * EB_SC
    assert _e % 8 + EB_SC <= ETAG_W and _e - _e % 8 + ETAG_W <= BATCH_N


def _sc_body(tags_ref, em_ref, start_ref, end_ref, trans_ref, out_ref,
             tags_v, etags_v, slab0, slab1, slab2, start_v, end_v, trans_v,
             acc_v, sem0, sem1, sem2):
    cid = lax.axis_index("c")
    sid = lax.axis_index("s")
    wid = sid * 2 + cid
    iota = lax.iota(jnp.int32, LANES)
    zero16 = jnp.zeros((LANES,), jnp.int32)

    eb0 = B_TC + wid * EB_SC
    slabs = (slab0, slab1, slab2)
    sems = (sem0, sem1, sem2)
    NBUF = len(slabs)
    copies = []
    if EB_SC:
        copies = [pltpu.async_copy(em_ref.at[eb0 + i], slabs[i], sems[i])
                  for i in range(min(NBUF, EB_SC))]

    pltpu.sync_copy(tags_ref.at[pl.ds(wid * B_PER_TILE, B_PER_TILE)], tags_v)
    e_off = lax.rem(eb0, 8)
    if EB_SC:
        e_base = pl.multiple_of(eb0 - e_off, 8)
        pltpu.sync_copy(tags_ref.at[pl.ds(e_base, ETAG_W)], etags_v)
    pltpu.sync_copy(start_ref, start_v)
    pltpu.sync_copy(end_ref, end_v)
    pltpu.sync_copy(trans_ref, trans_v)

    acc = jnp.zeros((LANES,), jnp.float32)
    for h in range(B_PER_TILE // LANES):
        brow = h * LANES + iota
        first = plsc.load_gather(tags_v, [brow, zero16])
        last = plsc.load_gather(tags_v, [brow, zero16 + (SEQ_N - 1)])
        acc = acc + plsc.load_gather(start_v, [first])
        acc = acc + plsc.load_gather(end_v, [last])

    def row_body(b, acc):
        brow = jnp.broadcast_to(b, (LANES,))
        cur0 = tags_v[b, pl.ds(0, LANES)]
        prev0 = plsc.load_gather(tags_v, [brow, jnp.maximum(iota - 1, 0)])
        tv0 = plsc.load_gather(trans_v, [prev0 * NUM_TAGS + cur0])
        acc = acc + jnp.where(iota == 0, jnp.zeros_like(tv0), tv0)
        for j in range(1, TVEC):
            cur = tags_v[b, pl.ds(j * LANES, LANES)]
            prev = plsc.load_gather(tags_v, [brow, j * LANES - 1 + iota])
            acc = acc + plsc.load_gather(trans_v, [prev * NUM_TAGS + cur])
        return acc

    for k in range(max(EB_SC, 1)):
        r_lo = k * B_PER_TILE // max(EB_SC, 1)
        r_hi = (k + 1) * B_PER_TILE // max(EB_SC, 1)
        acc = lax.fori_loop(r_lo, r_hi, row_body, acc)
        if not EB_SC:
            continue
        buf = slabs[k % NBUF]
        copies[k].wait()
        for j in range(TVEC):
            cur = etags_v[e_off + k, pl.ds(j * LANES, LANES)]
            acc = acc + plsc.load_gather(buf, [cur, j * LANES + iota])
        if k + NBUF < EB_SC:
            copies.append(
                pltpu.async_copy(em_ref.at[eb0 + k + NBUF], buf,
                                 sems[k % NBUF]))
        else:
            copies.append(None)

    acc_v[...] = acc
    pltpu.sync_copy(acc_v, out_ref.at[wid])


def _em_sum_body(tags_ref, em_ref, out_ref):
    i = pl.program_id(0)
    t_blk = tags_ref[...]
    em_blk = em_ref[...]
    g = lax.broadcasted_iota(jnp.int32, em_blk.shape, 1)
    s = jnp.sum(jnp.where(g == t_blk[:, None, :], em_blk, 0.0))

    @pl.when(i == 0)
    def _init():
        out_ref[0, 0] = 0.0

    out_ref[0, 0] += s


@jax.jit
def _crf_score(em, tags_i32, start, end, trans_flat):
    em_t = jnp.transpose(em, (0, 2, 1))

    sc_part = pl.kernel(
        _sc_body,
        out_type=jax.ShapeDtypeStruct((NUM_WORKERS, LANES), jnp.float32),
        mesh=plsc.VectorSubcoreMesh(core_axis_name="c", subcore_axis_name="s"),
        compiler_params=pltpu.CompilerParams(needs_layout_passes=False),
        scratch_types=[
            pltpu.VMEM((B_PER_TILE, SEQ_N), jnp.int32),
            pltpu.VMEM((max(ETAG_W, 1), SEQ_N), jnp.int32),
            pltpu.VMEM((NUM_TAGS, SEQ_N), jnp.float32),
            pltpu.VMEM((NUM_TAGS, SEQ_N), jnp.float32),
            pltpu.VMEM((NUM_TAGS, SEQ_N), jnp.float32),
            pltpu.VMEM((NUM_TAGS,), jnp.float32),
            pltpu.VMEM((NUM_TAGS,), jnp.float32),
            pltpu.VMEM((NUM_TAGS * NUM_TAGS,), jnp.float32),
            pltpu.VMEM((LANES,), jnp.float32),
            pltpu.SemaphoreType.DMA,
            pltpu.SemaphoreType.DMA,
            pltpu.SemaphoreType.DMA,
        ],
    )
    partials = sc_part(tags_i32, em_t, start, end, trans_flat)

    em_sum = pl.pallas_call(
        _em_sum_body,
        grid=(B_TC // EM_BB,),
        in_specs=[
            pl.BlockSpec((EM_BB, SEQ_N), lambda i: (i, 0)),
            pl.BlockSpec((EM_BB, NUM_TAGS, SEQ_N), lambda i: (i, 0, 0)),
        ],
        out_specs=pl.BlockSpec((1, 1), lambda i: (0, 0),
                               memory_space=pltpu.SMEM),
        out_shape=jax.ShapeDtypeStruct((1, 1), jnp.float32),
        compiler_params=pltpu.CompilerParams(
            dimension_semantics=("arbitrary",)),
    )(tags_i32, em_t)

    return jnp.sum(partials) + em_sum[0, 0]


def kernel(emissions, tags, mask, start_transitions, end_transitions, transitions):
    del mask
    total = _crf_score(emissions, tags.astype(jnp.int32), start_transitions,
                       end_transitions, transitions.reshape(-1))
    return total / BATCH_N
